# baseline (device time: 532587 ns/iter reference)
import functools

import numpy as np

import jax
import jax.numpy as jnp
from jax import lax
from jax.experimental import pallas as pl
from jax.experimental.pallas import tpu as pltpu

N_DEV = 4
SQ = 2048
D = 1024
DH = 128
H_LOC = 8
SCALE = 0.08838834764831843
LOG2E = 1.4426950408889634

_inv = 1.0 / (10000.0 ** (np.arange(0, DH, 2) / DH))
_pos = np.arange(SQ)[:, None] * _inv[None, :]
_COS = np.repeat(np.cos(_pos), 2, axis=-1).astype(np.float32)
_SIN = np.repeat(np.sin(_pos), 2, axis=-1).astype(np.float32)
_ROT = np.zeros((DH, DH), dtype=np.float32)
for _k in range(DH // 2):
    _ROT[2 * _k + 1, 2 * _k] = -1.0
    _ROT[2 * _k, 2 * _k + 1] = 1.0


def _attn_body(x_ref, wq_ref, wk_ref, wv_ref, cosq_ref, sinq_ref, cos_ref,
               sin_ref, rot_ref, out_ref, comm_ref, send_sems, recv_sems):
    b = pl.program_id(0)
    h = pl.program_id(1)
    my = lax.axis_index("i")
    left = lax.rem(my + 3, N_DEV)
    right = lax.rem(my + 1, N_DEV)

    @pl.when(jnp.logical_and(b == 0, h == 0))
    def _():
        barrier = pltpu.get_barrier_semaphore()
        for nbr in (left, right):
            pl.semaphore_signal(
                barrier, inc=1, device_id=(nbr,),
                device_id_type=pl.DeviceIdType.MESH,
            )
        pl.semaphore_wait(barrier, 2)
        comm_ref[N_DEV - 1, :, :] = x_ref[:]
        hop0 = pltpu.make_async_remote_copy(
            src_ref=x_ref,
            dst_ref=comm_ref.at[0],
            send_sem=send_sems.at[0],
            recv_sem=recv_sems.at[0],
            device_id=(right,),
            device_id_type=pl.DeviceIdType.MESH,
        )
        hop0.start()

    for bb in range(1, N_DEV):
        @pl.when(jnp.logical_and(b == bb, h == 0))
        def _(bb=bb):
            prev = pltpu.make_async_remote_copy(
                src_ref=comm_ref.at[bb - 1],
                dst_ref=comm_ref.at[bb - 1],
                send_sem=send_sems.at[bb - 1],
                recv_sem=recv_sems.at[bb - 1],
                device_id=(right,),
                device_id_type=pl.DeviceIdType.MESH,
            )
            prev.wait_send()
            prev.wait_recv()
            if bb <= N_DEV - 2:
                nxt = pltpu.make_async_remote_copy(
                    src_ref=comm_ref.at[bb - 1],
                    dst_ref=comm_ref.at[bb],
                    send_sem=send_sems.at[bb],
                    recv_sem=recv_sems.at[bb],
                    device_id=(right,),
                    device_id_type=pl.DeviceIdType.MESH,
                )
                nxt.start()

    xb = comm_ref[lax.rem(b + N_DEV - 1, N_DEV)]
    rot = rot_ref[:]

    q = jnp.dot(xb, wq_ref[:], preferred_element_type=jnp.float32)
    k = jnp.dot(xb, wk_ref[:], preferred_element_type=jnp.float32)
    v = jnp.dot(xb, wv_ref[:], preferred_element_type=jnp.float32)
    v = v.astype(jnp.bfloat16)

    qrot = jnp.dot(q.astype(jnp.bfloat16), rot,
                   preferred_element_type=jnp.float32)
    krot = jnp.dot(k.astype(jnp.bfloat16), rot,
                   preferred_element_type=jnp.float32)
    qr = (q * cosq_ref[:] + qrot * sinq_ref[:]).astype(jnp.bfloat16)
    kr = (k * cos_ref[:] + krot * sin_ref[:]).astype(jnp.bfloat16)

    s = lax.dot_general(qr, kr, (((1,), (1,)), ((), ())),
                        preferred_element_type=jnp.float32)
    w = jnp.exp2(s.astype(jnp.bfloat16))
    v_aug = jnp.concatenate([v, jnp.ones((SQ, DH), jnp.bfloat16)], axis=-1)
    ctx_aug = jnp.dot(w, v_aug, preferred_element_type=jnp.float32)
    ctx = ctx_aug[:, :DH]
    denom = ctx_aug[:, DH:DH + 1]
    out_ref[:] = (ctx * (1.0 / denom)).astype(jnp.bfloat16)


def _attention(x2, wq, wk, wv):
    cosq = jnp.asarray(_COS * (SCALE * LOG2E))
    sinq = jnp.asarray(_SIN * (SCALE * LOG2E))
    cos = jnp.asarray(_COS)
    sin = jnp.asarray(_SIN)
    rot = jnp.asarray(_ROT, dtype=jnp.bfloat16)
    return pl.pallas_call(
        _attn_body,
        grid=(N_DEV, H_LOC),
        in_specs=[
            pl.BlockSpec((SQ, D), lambda b, h: (0, 0)),
            pl.BlockSpec((D, DH), lambda b, h: (0, h)),
            pl.BlockSpec((D, DH), lambda b, h: (0, h)),
            pl.BlockSpec((D, DH), lambda b, h: (0, h)),
            pl.BlockSpec((SQ, DH), lambda b, h: (0, 0)),
            pl.BlockSpec((SQ, DH), lambda b, h: (0, 0)),
            pl.BlockSpec((SQ, DH), lambda b, h: (0, 0)),
            pl.BlockSpec((SQ, DH), lambda b, h: (0, 0)),
            pl.BlockSpec((DH, DH), lambda b, h: (0, 0)),
        ],
        out_specs=pl.BlockSpec((SQ, DH), lambda b, h: (b, h)),
        out_shape=jax.ShapeDtypeStruct((N_DEV * SQ, D), jnp.bfloat16),
        scratch_shapes=[
            pltpu.VMEM((N_DEV, SQ, D), jnp.bfloat16),
            pltpu.SemaphoreType.DMA((N_DEV - 1,)),
            pltpu.SemaphoreType.DMA((N_DEV - 1,)),
        ],
        compiler_params=pltpu.CompilerParams(
            collective_id=0, vmem_limit_bytes=60 * 1024 * 1024,
        ),
    )(x2, wq, wk, wv, cosq, sinq, cos, sin, rot)


def _rs_body(p_ref, wo_ref, out_ref, sbuf_ref, comm_ref, send_sems, recv_sems):
    my = lax.axis_index("i")
    left = lax.rem(my + 3, N_DEV)
    diag = lax.rem(my + 2, N_DEV)
    right = lax.rem(my + 1, N_DEV)
    targets = (left, diag, right)

    barrier = pltpu.get_barrier_semaphore()
    for nbr in targets:
        pl.semaphore_signal(
            barrier, inc=1, device_id=(nbr,),
            device_id_type=pl.DeviceIdType.MESH,
        )
    pl.semaphore_wait(barrier, 3)

    wo = wo_ref[:]
    rdmas = []
    for o in (1, 2, 3):
        proj = jnp.dot(p_ref[pl.ds(o * SQ, SQ), :], wo,
                       preferred_element_type=jnp.float32)
        sbuf_ref[o - 1, :, :] = proj.astype(jnp.bfloat16)
        rdma = pltpu.make_async_remote_copy(
            src_ref=sbuf_ref.at[o - 1],
            dst_ref=comm_ref.at[o - 1],
            send_sem=send_sems.at[o - 1],
            recv_sem=recv_sems.at[o - 1],
            device_id=(targets[o - 1],),
            device_id_type=pl.DeviceIdType.MESH,
        )
        rdma.start()
        rdmas.append(rdma)

    acc = jnp.dot(p_ref[pl.ds(0, SQ), :], wo,
                  preferred_element_type=jnp.float32)
    for rdma in rdmas:
        rdma.wait_send()
        rdma.wait_recv()
    for k in range(N_DEV - 1):
        acc = acc + comm_ref[k, :, :].astype(jnp.float32)
    out_ref[:] = acc


def _rs_proj(ctx, wo):
    return pl.pallas_call(
        _rs_body,
        out_shape=jax.ShapeDtypeStruct((SQ, D), jnp.float32),
        in_specs=[
            pl.BlockSpec(memory_space=pltpu.VMEM),
            pl.BlockSpec(memory_space=pltpu.VMEM),
        ],
        out_specs=pl.BlockSpec(memory_space=pltpu.VMEM),
        scratch_shapes=[
            pltpu.VMEM((N_DEV - 1, SQ, D), jnp.bfloat16),
            pltpu.VMEM((N_DEV - 1, SQ, D), jnp.bfloat16),
            pltpu.SemaphoreType.DMA((N_DEV - 1,)),
            pltpu.SemaphoreType.DMA((N_DEV - 1,)),
        ],
        compiler_params=pltpu.CompilerParams(
            collective_id=1, vmem_limit_bytes=62 * 1024 * 1024,
        ),
    )(ctx, wo)


def kernel(x, Wq, Wk, Wv, Wo):
    x2 = x.reshape(SQ, D).astype(jnp.bfloat16)
    ctx = _attention(
        x2,
        Wq.astype(jnp.bfloat16),
        Wk.astype(jnp.bfloat16),
        Wv.astype(jnp.bfloat16),
    )
    out = _rs_proj(ctx, Wo.astype(jnp.bfloat16))
    return out.reshape(1, SQ, D)


# device time: 521838 ns/iter; 1.0206x vs baseline; 1.0206x over previous
import functools

import numpy as np

import jax
import jax.numpy as jnp
from jax import lax
from jax.experimental import pallas as pl
from jax.experimental.pallas import tpu as pltpu

N_DEV = 4
SQ = 2048
D = 1024
DH = 128
H_LOC = 8
SCALE = 0.08838834764831843
LOG2E = 1.4426950408889634

_inv = 1.0 / (10000.0 ** (np.arange(0, DH, 2) / DH))
_pos = np.arange(SQ)[:, None] * _inv[None, :]
_COS = np.repeat(np.cos(_pos), 2, axis=-1).astype(np.float32)
_SIN = np.repeat(np.sin(_pos), 2, axis=-1).astype(np.float32)
_ROT = np.zeros((DH, DH), dtype=np.float32)
for _k in range(DH // 2):
    _ROT[2 * _k + 1, 2 * _k] = -1.0
    _ROT[2 * _k, 2 * _k + 1] = 1.0


def _attn_body(x_ref, wqkv_ref, cosq_ref, sinq_ref, cos_ref,
               sin_ref, rot_ref, out_ref, comm_ref, send_sems, recv_sems):
    b = pl.program_id(0)
    h = pl.program_id(1)
    my = lax.axis_index("i")
    left = lax.rem(my + 3, N_DEV)
    right = lax.rem(my + 1, N_DEV)

    @pl.when(jnp.logical_and(b == 0, h == 0))
    def _():
        barrier = pltpu.get_barrier_semaphore()
        for nbr in (left, right):
            pl.semaphore_signal(
                barrier, inc=1, device_id=(nbr,),
                device_id_type=pl.DeviceIdType.MESH,
            )
        pl.semaphore_wait(barrier, 2)
        comm_ref[N_DEV - 1, :, :] = x_ref[:]
        hop0 = pltpu.make_async_remote_copy(
            src_ref=x_ref,
            dst_ref=comm_ref.at[0],
            send_sem=send_sems.at[0],
            recv_sem=recv_sems.at[0],
            device_id=(right,),
            device_id_type=pl.DeviceIdType.MESH,
        )
        hop0.start()

    for bb in range(1, N_DEV):
        @pl.when(jnp.logical_and(b == bb, h == 0))
        def _(bb=bb):
            prev = pltpu.make_async_remote_copy(
                src_ref=comm_ref.at[bb - 1],
                dst_ref=comm_ref.at[bb - 1],
                send_sem=send_sems.at[bb - 1],
                recv_sem=recv_sems.at[bb - 1],
                device_id=(right,),
                device_id_type=pl.DeviceIdType.MESH,
            )
            prev.wait_send()
            prev.wait_recv()
            if bb <= N_DEV - 2:
                nxt = pltpu.make_async_remote_copy(
                    src_ref=comm_ref.at[bb - 1],
                    dst_ref=comm_ref.at[bb],
                    send_sem=send_sems.at[bb],
                    recv_sem=recv_sems.at[bb],
                    device_id=(right,),
                    device_id_type=pl.DeviceIdType.MESH,
                )
                nxt.start()

    xb = comm_ref[lax.rem(b + N_DEV - 1, N_DEV)]
    rot = rot_ref[:]

    qkv = jnp.dot(xb, wqkv_ref[:], preferred_element_type=jnp.float32)
    q = qkv[:, :DH]
    k = qkv[:, DH:2 * DH]
    v = qkv[:, 2 * DH:].astype(jnp.bfloat16)

    qrot = jnp.dot(q.astype(jnp.bfloat16), rot,
                   preferred_element_type=jnp.float32)
    krot = jnp.dot(k.astype(jnp.bfloat16), rot,
                   preferred_element_type=jnp.float32)
    qr = (q * cosq_ref[:] + qrot * sinq_ref[:]).astype(jnp.bfloat16)
    kr = (k * cos_ref[:] + krot * sin_ref[:]).astype(jnp.bfloat16)

    s = lax.dot_general(qr, kr, (((1,), (1,)), ((), ())),
                        preferred_element_type=jnp.float32)
    w = jnp.exp2(s.astype(jnp.bfloat16))
    v_aug = jnp.concatenate([v, jnp.ones((SQ, DH), jnp.bfloat16)], axis=-1)
    ctx_aug = jnp.dot(w, v_aug, preferred_element_type=jnp.float32)
    ctx = ctx_aug[:, :DH]
    denom = ctx_aug[:, DH:DH + 1]
    out_ref[:] = (ctx * (1.0 / denom)).astype(jnp.bfloat16)


def _attention(x2, wqkv):
    cosq = jnp.asarray(_COS * (SCALE * LOG2E))
    sinq = jnp.asarray(_SIN * (SCALE * LOG2E))
    cos = jnp.asarray(_COS)
    sin = jnp.asarray(_SIN)
    rot = jnp.asarray(_ROT, dtype=jnp.bfloat16)
    return pl.pallas_call(
        _attn_body,
        grid=(N_DEV, H_LOC),
        in_specs=[
            pl.BlockSpec((SQ, D), lambda b, h: (0, 0)),
            pl.BlockSpec((D, 3 * DH), lambda b, h: (0, h)),
            pl.BlockSpec((SQ, DH), lambda b, h: (0, 0)),
            pl.BlockSpec((SQ, DH), lambda b, h: (0, 0)),
            pl.BlockSpec((SQ, DH), lambda b, h: (0, 0)),
            pl.BlockSpec((SQ, DH), lambda b, h: (0, 0)),
            pl.BlockSpec((DH, DH), lambda b, h: (0, 0)),
        ],
        out_specs=pl.BlockSpec((SQ, DH), lambda b, h: (b, h)),
        out_shape=jax.ShapeDtypeStruct((N_DEV * SQ, D), jnp.bfloat16),
        scratch_shapes=[
            pltpu.VMEM((N_DEV, SQ, D), jnp.bfloat16),
            pltpu.SemaphoreType.DMA((N_DEV - 1,)),
            pltpu.SemaphoreType.DMA((N_DEV - 1,)),
        ],
        compiler_params=pltpu.CompilerParams(
            collective_id=0, vmem_limit_bytes=60 * 1024 * 1024,
        ),
    )(x2, wqkv, cosq, sinq, cos, sin, rot)


def _rs_body(p_ref, wo_ref, out_ref, sbuf_ref, comm_ref, send_sems, recv_sems):
    my = lax.axis_index("i")
    left = lax.rem(my + 3, N_DEV)
    diag = lax.rem(my + 2, N_DEV)
    right = lax.rem(my + 1, N_DEV)
    targets = (left, diag, right)

    barrier = pltpu.get_barrier_semaphore()
    for nbr in targets:
        pl.semaphore_signal(
            barrier, inc=1, device_id=(nbr,),
            device_id_type=pl.DeviceIdType.MESH,
        )
    pl.semaphore_wait(barrier, 3)

    wo = wo_ref[:]
    rdmas = []
    for o in (1, 2, 3):
        proj = jnp.dot(p_ref[pl.ds(o * SQ, SQ), :], wo,
                       preferred_element_type=jnp.float32)
        sbuf_ref[o - 1, :, :] = proj.astype(jnp.bfloat16)
        rdma = pltpu.make_async_remote_copy(
            src_ref=sbuf_ref.at[o - 1],
            dst_ref=comm_ref.at[o - 1],
            send_sem=send_sems.at[o - 1],
            recv_sem=recv_sems.at[o - 1],
            device_id=(targets[o - 1],),
            device_id_type=pl.DeviceIdType.MESH,
        )
        rdma.start()
        rdmas.append(rdma)

    acc = jnp.dot(p_ref[pl.ds(0, SQ), :], wo,
                  preferred_element_type=jnp.float32)
    for rdma in rdmas:
        rdma.wait_send()
        rdma.wait_recv()
    for k in range(N_DEV - 1):
        acc = acc + comm_ref[k, :, :].astype(jnp.float32)
    out_ref[:] = acc


def _rs_proj(ctx, wo):
    return pl.pallas_call(
        _rs_body,
        out_shape=jax.ShapeDtypeStruct((SQ, D), jnp.float32),
        in_specs=[
            pl.BlockSpec(memory_space=pltpu.VMEM),
            pl.BlockSpec(memory_space=pltpu.VMEM),
        ],
        out_specs=pl.BlockSpec(memory_space=pltpu.VMEM),
        scratch_shapes=[
            pltpu.VMEM((N_DEV - 1, SQ, D), jnp.bfloat16),
            pltpu.VMEM((N_DEV - 1, SQ, D), jnp.bfloat16),
            pltpu.SemaphoreType.DMA((N_DEV - 1,)),
            pltpu.SemaphoreType.DMA((N_DEV - 1,)),
        ],
        compiler_params=pltpu.CompilerParams(
            collective_id=1, vmem_limit_bytes=62 * 1024 * 1024,
        ),
    )(ctx, wo)


def kernel(x, Wq, Wk, Wv, Wo):
    x2 = x.reshape(SQ, D).astype(jnp.bfloat16)
    wqkv = jnp.concatenate(
        [
            Wq.astype(jnp.bfloat16).reshape(D, H_LOC, DH),
            Wk.astype(jnp.bfloat16).reshape(D, H_LOC, DH),
            Wv.astype(jnp.bfloat16).reshape(D, H_LOC, DH),
        ],
        axis=2,
    ).reshape(D, H_LOC * 3 * DH)
    ctx = _attention(x2, wqkv)
    out = _rs_proj(ctx, Wo.astype(jnp.bfloat16))
    return out.reshape(1, SQ, D)


# device time: 392376 ns/iter; 1.3573x vs baseline; 1.3299x over previous
import functools

import numpy as np

import jax
import jax.numpy as jnp
from jax import lax
from jax.experimental import pallas as pl
from jax.experimental.pallas import tpu as pltpu

N_DEV = 4
SQ = 2048
D = 1024
DH = 128
H_LOC = 8
SCALE = 0.08838834764831843
LOG2E = 1.4426950408889634

_inv = 1.0 / (10000.0 ** (np.arange(0, DH, 2) / DH))
_pos = np.arange(SQ)[:, None] * _inv[None, :]
_COS = np.repeat(np.cos(_pos), 2, axis=-1).astype(np.float32)
_SIN = np.repeat(np.sin(_pos), 2, axis=-1).astype(np.float32)
_ROT = np.zeros((DH, DH), dtype=np.float32)
for _k in range(DH // 2):
    _ROT[2 * _k + 1, 2 * _k] = -1.0
    _ROT[2 * _k, 2 * _k + 1] = 1.0


def _attn_body(x_ref, wqkv_ref, cosq_ref, sinq_ref, cos_ref,
               sin_ref, rot_ref, out_ref, xb_ref, comm_ref, send_sems,
               recv_sems):
    b = pl.program_id(0)
    h = pl.program_id(1)
    my = lax.axis_index("i")
    left = lax.rem(my + 3, N_DEV)
    right = lax.rem(my + 1, N_DEV)

    @pl.when(jnp.logical_and(b == 0, h == 0))
    def _():
        barrier = pltpu.get_barrier_semaphore()
        for nbr in (left, right):
            pl.semaphore_signal(
                barrier, inc=1, device_id=(nbr,),
                device_id_type=pl.DeviceIdType.MESH,
            )
        pl.semaphore_wait(barrier, 2)
        comm_ref[N_DEV - 1, :, :] = x_ref[:]
        hop0 = pltpu.make_async_remote_copy(
            src_ref=x_ref,
            dst_ref=comm_ref.at[0],
            send_sem=send_sems.at[0],
            recv_sem=recv_sems.at[0],
            device_id=(right,),
            device_id_type=pl.DeviceIdType.MESH,
        )
        hop0.start()

    for bb in range(1, N_DEV):
        @pl.when(jnp.logical_and(b == bb, h == 0))
        def _(bb=bb):
            prev = pltpu.make_async_remote_copy(
                src_ref=comm_ref.at[bb - 1],
                dst_ref=comm_ref.at[bb - 1],
                send_sem=send_sems.at[bb - 1],
                recv_sem=recv_sems.at[bb - 1],
                device_id=(right,),
                device_id_type=pl.DeviceIdType.MESH,
            )
            prev.wait_send()
            prev.wait_recv()
            if bb <= N_DEV - 2:
                nxt = pltpu.make_async_remote_copy(
                    src_ref=comm_ref.at[bb - 1],
                    dst_ref=comm_ref.at[bb],
                    send_sem=send_sems.at[bb],
                    recv_sem=recv_sems.at[bb],
                    device_id=(right,),
                    device_id_type=pl.DeviceIdType.MESH,
                )
                nxt.start()

    @pl.when(h == 0)
    def _():
        xb_ref[:] = comm_ref[lax.rem(b + N_DEV - 1, N_DEV)]

    xb = xb_ref[:]
    rot = rot_ref[:]

    qkv = jnp.dot(xb, wqkv_ref[:], preferred_element_type=jnp.float32)
    q = qkv[:, :DH]
    k = qkv[:, DH:2 * DH]
    v = qkv[:, 2 * DH:].astype(jnp.bfloat16)

    qrot = jnp.dot(q.astype(jnp.bfloat16), rot,
                   preferred_element_type=jnp.float32)
    krot = jnp.dot(k.astype(jnp.bfloat16), rot,
                   preferred_element_type=jnp.float32)
    qr = (q * cosq_ref[:] + qrot * sinq_ref[:]).astype(jnp.bfloat16)
    kr = (k * cos_ref[:] + krot * sin_ref[:]).astype(jnp.bfloat16)

    v_aug = jnp.concatenate([v, jnp.ones((SQ, DH), jnp.bfloat16)], axis=-1)
    TQ = SQ // 2
    s0 = lax.dot_general(qr[:TQ], kr, (((1,), (1,)), ((), ())),
                         preferred_element_type=jnp.float32)
    s1 = lax.dot_general(qr[TQ:], kr, (((1,), (1,)), ((), ())),
                         preferred_element_type=jnp.float32)
    w0 = jnp.exp2(s0.astype(jnp.bfloat16))
    ctx0 = jnp.dot(w0, v_aug, preferred_element_type=jnp.float32)
    w1 = jnp.exp2(s1.astype(jnp.bfloat16))
    ctx1 = jnp.dot(w1, v_aug, preferred_element_type=jnp.float32)
    out_ref[:TQ] = (ctx0[:, :DH] * (1.0 / ctx0[:, DH:DH + 1])).astype(
        jnp.bfloat16)
    out_ref[TQ:] = (ctx1[:, :DH] * (1.0 / ctx1[:, DH:DH + 1])).astype(
        jnp.bfloat16)


def _attention(x2, wqkv):
    cosq = jnp.asarray(_COS * (SCALE * LOG2E))
    sinq = jnp.asarray(_SIN * (SCALE * LOG2E))
    cos = jnp.asarray(_COS)
    sin = jnp.asarray(_SIN)
    rot = jnp.asarray(_ROT, dtype=jnp.bfloat16)
    return pl.pallas_call(
        _attn_body,
        grid=(N_DEV, H_LOC),
        in_specs=[
            pl.BlockSpec((SQ, D), lambda b, h: (0, 0)),
            pl.BlockSpec((D, 3 * DH), lambda b, h: (0, h)),
            pl.BlockSpec((SQ, DH), lambda b, h: (0, 0)),
            pl.BlockSpec((SQ, DH), lambda b, h: (0, 0)),
            pl.BlockSpec((SQ, DH), lambda b, h: (0, 0)),
            pl.BlockSpec((SQ, DH), lambda b, h: (0, 0)),
            pl.BlockSpec((DH, DH), lambda b, h: (0, 0)),
        ],
        out_specs=pl.BlockSpec((SQ, DH), lambda b, h: (b, h)),
        out_shape=jax.ShapeDtypeStruct((N_DEV * SQ, D), jnp.bfloat16),
        scratch_shapes=[
            pltpu.VMEM((SQ, D), jnp.bfloat16),
            pltpu.VMEM((N_DEV, SQ, D), jnp.bfloat16),
            pltpu.SemaphoreType.DMA((N_DEV - 1,)),
            pltpu.SemaphoreType.DMA((N_DEV - 1,)),
        ],
        compiler_params=pltpu.CompilerParams(
            collective_id=0, vmem_limit_bytes=62 * 1024 * 1024,
        ),
    )(x2, wqkv, cosq, sinq, cos, sin, rot)


def _rs_body(p_ref, wo_ref, out_ref, sbuf_ref, comm_ref, send_sems, recv_sems):
    my = lax.axis_index("i")
    left = lax.rem(my + 3, N_DEV)
    diag = lax.rem(my + 2, N_DEV)
    right = lax.rem(my + 1, N_DEV)
    targets = (left, diag, right)

    barrier = pltpu.get_barrier_semaphore()
    for nbr in targets:
        pl.semaphore_signal(
            barrier, inc=1, device_id=(nbr,),
            device_id_type=pl.DeviceIdType.MESH,
        )
    pl.semaphore_wait(barrier, 3)

    wo = wo_ref[:]
    rdmas = []
    for o in (1, 2, 3):
        proj = jnp.dot(p_ref[pl.ds(o * SQ, SQ), :], wo,
                       preferred_element_type=jnp.float32)
        sbuf_ref[o - 1, :, :] = proj.astype(jnp.bfloat16)
        rdma = pltpu.make_async_remote_copy(
            src_ref=sbuf_ref.at[o - 1],
            dst_ref=comm_ref.at[o - 1],
            send_sem=send_sems.at[o - 1],
            recv_sem=recv_sems.at[o - 1],
            device_id=(targets[o - 1],),
            device_id_type=pl.DeviceIdType.MESH,
        )
        rdma.start()
        rdmas.append(rdma)

    acc = jnp.dot(p_ref[pl.ds(0, SQ), :], wo,
                  preferred_element_type=jnp.float32)
    for rdma in rdmas:
        rdma.wait_send()
        rdma.wait_recv()
    for k in range(N_DEV - 1):
        acc = acc + comm_ref[k, :, :].astype(jnp.float32)
    out_ref[:] = acc


def _rs_proj(ctx, wo):
    return pl.pallas_call(
        _rs_body,
        out_shape=jax.ShapeDtypeStruct((SQ, D), jnp.float32),
        in_specs=[
            pl.BlockSpec(memory_space=pltpu.VMEM),
            pl.BlockSpec(memory_space=pltpu.VMEM),
        ],
        out_specs=pl.BlockSpec(memory_space=pltpu.VMEM),
        scratch_shapes=[
            pltpu.VMEM((N_DEV - 1, SQ, D), jnp.bfloat16),
            pltpu.VMEM((N_DEV - 1, SQ, D), jnp.bfloat16),
            pltpu.SemaphoreType.DMA((N_DEV - 1,)),
            pltpu.SemaphoreType.DMA((N_DEV - 1,)),
        ],
        compiler_params=pltpu.CompilerParams(
            collective_id=1, vmem_limit_bytes=62 * 1024 * 1024,
        ),
    )(ctx, wo)


def kernel(x, Wq, Wk, Wv, Wo):
    x2 = x.reshape(SQ, D).astype(jnp.bfloat16)
    wqkv = jnp.concatenate(
        [
            Wq.astype(jnp.bfloat16).reshape(D, H_LOC, DH),
            Wk.astype(jnp.bfloat16).reshape(D, H_LOC, DH),
            Wv.astype(jnp.bfloat16).reshape(D, H_LOC, DH),
        ],
        axis=2,
    ).reshape(D, H_LOC * 3 * DH)
    ctx = _attention(x2, wqkv)
    out = _rs_proj(ctx, Wo.astype(jnp.bfloat16))
    return out.reshape(1, SQ, D)


# device time: 388312 ns/iter; 1.3715x vs baseline; 1.0105x over previous
import functools

import numpy as np

import jax
import jax.numpy as jnp
from jax import lax
from jax.experimental import pallas as pl
from jax.experimental.pallas import tpu as pltpu

N_DEV = 4
SQ = 2048
D = 1024
DH = 128
H_LOC = 8
SCALE = 0.08838834764831843
LOG2E = 1.4426950408889634

_inv = 1.0 / (10000.0 ** (np.arange(0, DH, 2) / DH))
_pos = np.arange(SQ)[:, None] * _inv[None, :]
_COS = np.repeat(np.cos(_pos), 2, axis=-1).astype(np.float32)
_SIN = np.repeat(np.sin(_pos), 2, axis=-1).astype(np.float32)
_ROT = np.zeros((DH, DH), dtype=np.float32)
for _k in range(DH // 2):
    _ROT[2 * _k + 1, 2 * _k] = -1.0
    _ROT[2 * _k, 2 * _k + 1] = 1.0


def _attn_body(x_ref, wqkv_ref, cosq_ref, sinq_ref, cos_ref,
               sin_ref, rot_ref, out_ref, xb_ref, comm_ref, send_sems,
               recv_sems):
    b = pl.program_id(0)
    h = pl.program_id(1)
    my = lax.axis_index("i")
    left = lax.rem(my + 3, N_DEV)
    right = lax.rem(my + 1, N_DEV)

    @pl.when(jnp.logical_and(b == 0, h == 0))
    def _():
        barrier = pltpu.get_barrier_semaphore()
        for nbr in (left, right):
            pl.semaphore_signal(
                barrier, inc=1, device_id=(nbr,),
                device_id_type=pl.DeviceIdType.MESH,
            )
        pl.semaphore_wait(barrier, 2)
        comm_ref[N_DEV - 1, :, :] = x_ref[:]
        hop0 = pltpu.make_async_remote_copy(
            src_ref=x_ref,
            dst_ref=comm_ref.at[0],
            send_sem=send_sems.at[0],
            recv_sem=recv_sems.at[0],
            device_id=(right,),
            device_id_type=pl.DeviceIdType.MESH,
        )
        hop0.start()

    for bb in range(1, N_DEV):
        @pl.when(jnp.logical_and(b == bb, h == 0))
        def _(bb=bb):
            prev = pltpu.make_async_remote_copy(
                src_ref=comm_ref.at[bb - 1],
                dst_ref=comm_ref.at[bb - 1],
                send_sem=send_sems.at[bb - 1],
                recv_sem=recv_sems.at[bb - 1],
                device_id=(right,),
                device_id_type=pl.DeviceIdType.MESH,
            )
            prev.wait_send()
            prev.wait_recv()
            if bb <= N_DEV - 2:
                nxt = pltpu.make_async_remote_copy(
                    src_ref=comm_ref.at[bb - 1],
                    dst_ref=comm_ref.at[bb],
                    send_sem=send_sems.at[bb],
                    recv_sem=recv_sems.at[bb],
                    device_id=(right,),
                    device_id_type=pl.DeviceIdType.MESH,
                )
                nxt.start()

    @pl.when(h == 0)
    def _():
        xb_ref[:] = comm_ref[lax.rem(b + N_DEV - 1, N_DEV)]

    xb = xb_ref[:]
    rot = rot_ref[:]

    qkv = jnp.dot(xb, wqkv_ref[:], preferred_element_type=jnp.float32)
    q = qkv[:, :DH]
    k = qkv[:, DH:2 * DH]
    v = qkv[:, 2 * DH:].astype(jnp.bfloat16)

    qrot = jnp.dot(q.astype(jnp.bfloat16), rot,
                   preferred_element_type=jnp.float32)
    krot = jnp.dot(k.astype(jnp.bfloat16), rot,
                   preferred_element_type=jnp.float32)
    qr = (q * cosq_ref[:] + qrot * sinq_ref[:]).astype(jnp.bfloat16)
    kr = (k * cos_ref[:] + krot * sin_ref[:]).astype(jnp.bfloat16)

    v_aug = jnp.concatenate([v, jnp.ones((SQ, DH), jnp.bfloat16)], axis=-1)
    TQ = SQ // 2
    s0 = lax.dot_general(qr[:TQ], kr, (((1,), (1,)), ((), ())),
                         preferred_element_type=jnp.float32)
    s1 = lax.dot_general(qr[TQ:], kr, (((1,), (1,)), ((), ())),
                         preferred_element_type=jnp.float32)
    w0 = jnp.exp2(s0.astype(jnp.bfloat16))
    ctx0 = jnp.dot(w0, v_aug, preferred_element_type=jnp.float32)
    w1 = jnp.exp2(s1.astype(jnp.bfloat16))
    ctx1 = jnp.dot(w1, v_aug, preferred_element_type=jnp.float32)
    out_ref[:TQ] = (ctx0[:, :DH] * (1.0 / ctx0[:, DH:DH + 1])).astype(
        jnp.bfloat16)
    out_ref[TQ:] = (ctx1[:, :DH] * (1.0 / ctx1[:, DH:DH + 1])).astype(
        jnp.bfloat16)


def _attention(x2, wqkv):
    cosq = jnp.asarray(_COS * (SCALE * LOG2E))
    sinq = jnp.asarray(_SIN * (SCALE * LOG2E))
    cos = jnp.asarray(_COS)
    sin = jnp.asarray(_SIN)
    rot = jnp.asarray(_ROT, dtype=jnp.bfloat16)
    return pl.pallas_call(
        _attn_body,
        grid=(N_DEV, H_LOC),
        in_specs=[
            pl.BlockSpec((SQ, D), lambda b, h: (0, 0)),
            pl.BlockSpec((D, 3 * DH), lambda b, h: (0, h)),
            pl.BlockSpec((SQ, DH), lambda b, h: (0, 0)),
            pl.BlockSpec((SQ, DH), lambda b, h: (0, 0)),
            pl.BlockSpec((SQ, DH), lambda b, h: (0, 0)),
            pl.BlockSpec((SQ, DH), lambda b, h: (0, 0)),
            pl.BlockSpec((DH, DH), lambda b, h: (0, 0)),
        ],
        out_specs=pl.BlockSpec((SQ, DH), lambda b, h: (b, h)),
        out_shape=jax.ShapeDtypeStruct((N_DEV * SQ, D), jnp.bfloat16),
        scratch_shapes=[
            pltpu.VMEM((SQ, D), jnp.bfloat16),
            pltpu.VMEM((N_DEV, SQ, D), jnp.bfloat16),
            pltpu.SemaphoreType.DMA((N_DEV - 1,)),
            pltpu.SemaphoreType.DMA((N_DEV - 1,)),
        ],
        compiler_params=pltpu.CompilerParams(
            collective_id=0, vmem_limit_bytes=62 * 1024 * 1024,
        ),
    )(x2, wqkv, cosq, sinq, cos, sin, rot)


def _rs_body(p_ref, wo_ref, out_ref, sbuf_ref, comm_ref, send_sems, recv_sems):
    my = lax.axis_index("i")
    left = lax.rem(my + 3, N_DEV)
    diag = lax.rem(my + 2, N_DEV)
    right = lax.rem(my + 1, N_DEV)
    targets = (left, diag, right)

    barrier = pltpu.get_barrier_semaphore()
    for nbr in targets:
        pl.semaphore_signal(
            barrier, inc=1, device_id=(nbr,),
            device_id_type=pl.DeviceIdType.MESH,
        )
    pl.semaphore_wait(barrier, 3)

    wo = wo_ref[:]
    rdmas = []
    for o in (2, 1, 3):
        proj = jnp.dot(p_ref[pl.ds(o * SQ, SQ), :], wo,
                       preferred_element_type=jnp.float32)
        sbuf_ref[o - 1, :, :] = proj.astype(jnp.bfloat16)
        rdma = pltpu.make_async_remote_copy(
            src_ref=sbuf_ref.at[o - 1],
            dst_ref=comm_ref.at[o - 1],
            send_sem=send_sems.at[o - 1],
            recv_sem=recv_sems.at[o - 1],
            device_id=(targets[o - 1],),
            device_id_type=pl.DeviceIdType.MESH,
        )
        rdma.start()
        rdmas.append(rdma)

    acc = jnp.dot(p_ref[pl.ds(0, SQ), :], wo,
                  preferred_element_type=jnp.float32)
    for rdma in rdmas:
        rdma.wait_send()
        rdma.wait_recv()
    for k in range(N_DEV - 1):
        acc = acc + comm_ref[k, :, :].astype(jnp.float32)
    out_ref[:] = acc


def _rs_proj(ctx, wo):
    return pl.pallas_call(
        _rs_body,
        out_shape=jax.ShapeDtypeStruct((SQ, D), jnp.float32),
        in_specs=[
            pl.BlockSpec(memory_space=pltpu.VMEM),
            pl.BlockSpec(memory_space=pltpu.VMEM),
        ],
        out_specs=pl.BlockSpec(memory_space=pltpu.VMEM),
        scratch_shapes=[
            pltpu.VMEM((N_DEV - 1, SQ, D), jnp.bfloat16),
            pltpu.VMEM((N_DEV - 1, SQ, D), jnp.bfloat16),
            pltpu.SemaphoreType.DMA((N_DEV - 1,)),
            pltpu.SemaphoreType.DMA((N_DEV - 1,)),
        ],
        compiler_params=pltpu.CompilerParams(
            collective_id=1, vmem_limit_bytes=62 * 1024 * 1024,
        ),
    )(ctx, wo)


def kernel(x, Wq, Wk, Wv, Wo):
    x2 = x.reshape(SQ, D).astype(jnp.bfloat16)
    wqkv = jnp.concatenate(
        [
            Wq.astype(jnp.bfloat16).reshape(D, H_LOC, DH),
            Wk.astype(jnp.bfloat16).reshape(D, H_LOC, DH),
            Wv.astype(jnp.bfloat16).reshape(D, H_LOC, DH),
        ],
        axis=2,
    ).reshape(D, H_LOC * 3 * DH)
    ctx = _attention(x2, wqkv)
    out = _rs_proj(ctx, Wo.astype(jnp.bfloat16))
    return out.reshape(1, SQ, D)


# device time: 386271 ns/iter; 1.3788x vs baseline; 1.0053x over previous
import functools

import numpy as np

import jax
import jax.numpy as jnp
from jax import lax
from jax.experimental import pallas as pl
from jax.experimental.pallas import tpu as pltpu

N_DEV = 4
SQ = 2048
D = 1024
DH = 128
H_LOC = 8
SCALE = 0.08838834764831843
LOG2E = 1.4426950408889634

_inv = 1.0 / (10000.0 ** (np.arange(0, DH, 2) / DH))
_pos = np.arange(SQ)[:, None] * _inv[None, :]
_COS = np.repeat(np.cos(_pos), 2, axis=-1).astype(np.float32)
_SIN = np.repeat(np.sin(_pos), 2, axis=-1).astype(np.float32)
_ROT = np.zeros((DH, DH), dtype=np.float32)
for _k in range(DH // 2):
    _ROT[2 * _k + 1, 2 * _k] = -1.0
    _ROT[2 * _k, 2 * _k + 1] = 1.0


def _attn_body(x_ref, wqkv_ref, cosq_ref, sinq_ref, cos_ref,
               sin_ref, rot_ref, out_ref, xb_ref, comm_ref, send_sems,
               recv_sems):
    b = pl.program_id(0)
    h = pl.program_id(1)
    my = lax.axis_index("i")
    left = lax.rem(my + 3, N_DEV)
    right = lax.rem(my + 1, N_DEV)

    @pl.when(jnp.logical_and(b == 0, h == 0))
    def _():
        barrier = pltpu.get_barrier_semaphore()
        for nbr in (left, right):
            pl.semaphore_signal(
                barrier, inc=1, device_id=(nbr,),
                device_id_type=pl.DeviceIdType.MESH,
            )
        pl.semaphore_wait(barrier, 2)
        comm_ref[N_DEV - 1, :, :] = x_ref[:]
        hop0 = pltpu.make_async_remote_copy(
            src_ref=x_ref,
            dst_ref=comm_ref.at[0],
            send_sem=send_sems.at[0],
            recv_sem=recv_sems.at[0],
            device_id=(right,),
            device_id_type=pl.DeviceIdType.MESH,
        )
        hop0.start()

    for bb in range(1, N_DEV):
        @pl.when(jnp.logical_and(b == bb, h == 0))
        def _(bb=bb):
            prev = pltpu.make_async_remote_copy(
                src_ref=comm_ref.at[bb - 1],
                dst_ref=comm_ref.at[bb - 1],
                send_sem=send_sems.at[bb - 1],
                recv_sem=recv_sems.at[bb - 1],
                device_id=(right,),
                device_id_type=pl.DeviceIdType.MESH,
            )
            prev.wait_send()
            prev.wait_recv()
            if bb <= N_DEV - 2:
                nxt = pltpu.make_async_remote_copy(
                    src_ref=comm_ref.at[bb - 1],
                    dst_ref=comm_ref.at[bb],
                    send_sem=send_sems.at[bb],
                    recv_sem=recv_sems.at[bb],
                    device_id=(right,),
                    device_id_type=pl.DeviceIdType.MESH,
                )
                nxt.start()

    @pl.when(h == 0)
    def _():
        xb_ref[:] = comm_ref[lax.rem(b + N_DEV - 1, N_DEV)]

    xb = xb_ref[:]
    rot = rot_ref[:]

    qkv = jnp.dot(xb, wqkv_ref[:], preferred_element_type=jnp.float32)
    q = qkv[:, :DH]
    k = qkv[:, DH:2 * DH]
    v = qkv[:, 2 * DH:].astype(jnp.bfloat16)

    qrot = jnp.dot(q.astype(jnp.bfloat16), rot,
                   preferred_element_type=jnp.float32)
    krot = jnp.dot(k.astype(jnp.bfloat16), rot,
                   preferred_element_type=jnp.float32)
    qr = (q * cosq_ref[:] + qrot * sinq_ref[:]).astype(jnp.bfloat16)
    kr = (k * cos_ref[:] + krot * sin_ref[:]).astype(jnp.bfloat16)

    v_aug = jnp.concatenate([v, jnp.ones((SQ, DH), jnp.bfloat16)], axis=-1)
    TQ = SQ // 2
    s0 = lax.dot_general(qr[:TQ], kr, (((1,), (1,)), ((), ())),
                         preferred_element_type=jnp.float32)
    s1 = lax.dot_general(qr[TQ:], kr, (((1,), (1,)), ((), ())),
                         preferred_element_type=jnp.float32)
    w0 = jnp.exp2(s0.astype(jnp.bfloat16))
    ctx0 = jnp.dot(w0, v_aug, preferred_element_type=jnp.float32)
    w1 = jnp.exp2(s1.astype(jnp.bfloat16))
    ctx1 = jnp.dot(w1, v_aug, preferred_element_type=jnp.float32)
    out_ref[:TQ] = (ctx0[:, :DH] * (1.0 / ctx0[:, DH:DH + 1])).astype(
        jnp.bfloat16)
    out_ref[TQ:] = (ctx1[:, :DH] * (1.0 / ctx1[:, DH:DH + 1])).astype(
        jnp.bfloat16)


def _attention(x2, wqkv):
    cosq = jnp.asarray(_COS * (SCALE * LOG2E))
    sinq = jnp.asarray(_SIN * (SCALE * LOG2E))
    cos = jnp.asarray(_COS)
    sin = jnp.asarray(_SIN)
    rot = jnp.asarray(_ROT, dtype=jnp.bfloat16)
    return pl.pallas_call(
        _attn_body,
        grid=(N_DEV, H_LOC),
        in_specs=[
            pl.BlockSpec((SQ, D), lambda b, h: (0, 0)),
            pl.BlockSpec((D, 3 * DH), lambda b, h: (0, h)),
            pl.BlockSpec((SQ, DH), lambda b, h: (0, 0)),
            pl.BlockSpec((SQ, DH), lambda b, h: (0, 0)),
            pl.BlockSpec((SQ, DH), lambda b, h: (0, 0)),
            pl.BlockSpec((SQ, DH), lambda b, h: (0, 0)),
            pl.BlockSpec((DH, DH), lambda b, h: (0, 0)),
        ],
        out_specs=pl.BlockSpec((SQ, DH), lambda b, h: (b, h)),
        out_shape=jax.ShapeDtypeStruct((N_DEV * SQ, D), jnp.bfloat16),
        scratch_shapes=[
            pltpu.VMEM((SQ, D), jnp.bfloat16),
            pltpu.VMEM((N_DEV, SQ, D), jnp.bfloat16),
            pltpu.SemaphoreType.DMA((N_DEV - 1,)),
            pltpu.SemaphoreType.DMA((N_DEV - 1,)),
        ],
        compiler_params=pltpu.CompilerParams(
            collective_id=0, vmem_limit_bytes=62 * 1024 * 1024,
        ),
    )(x2, wqkv, cosq, sinq, cos, sin, rot)


def _rs_body(p_ref, wo_ref, out_ref, sbuf_ref, comm_ref, send_sems, recv_sems):
    my = lax.axis_index("i")
    left = lax.rem(my + 3, N_DEV)
    diag = lax.rem(my + 2, N_DEV)
    right = lax.rem(my + 1, N_DEV)
    targets = (left, diag, right)

    barrier = pltpu.get_barrier_semaphore()
    for nbr in targets:
        pl.semaphore_signal(
            barrier, inc=1, device_id=(nbr,),
            device_id_type=pl.DeviceIdType.MESH,
        )
    pl.semaphore_wait(barrier, 3)

    wo = wo_ref[:]
    HQ = SQ // 2
    rdmas = []
    for o in (2, 1, 3):
        for hh in range(2):
            proj = jnp.dot(p_ref[pl.ds(o * SQ + hh * HQ, HQ), :], wo,
                           preferred_element_type=jnp.float32)
            sbuf_ref[o - 1, pl.ds(hh * HQ, HQ), :] = proj.astype(jnp.bfloat16)
            rdma = pltpu.make_async_remote_copy(
                src_ref=sbuf_ref.at[o - 1, pl.ds(hh * HQ, HQ), :],
                dst_ref=comm_ref.at[o - 1, pl.ds(hh * HQ, HQ), :],
                send_sem=send_sems.at[o - 1, hh],
                recv_sem=recv_sems.at[o - 1, hh],
                device_id=(targets[o - 1],),
                device_id_type=pl.DeviceIdType.MESH,
            )
            rdma.start()
            rdmas.append(rdma)

    acc = jnp.dot(p_ref[pl.ds(0, SQ), :], wo,
                  preferred_element_type=jnp.float32)
    for rdma in rdmas:
        rdma.wait_send()
        rdma.wait_recv()
    for k in range(N_DEV - 1):
        acc = acc + comm_ref[k, :, :].astype(jnp.float32)
    out_ref[:] = acc


def _rs_proj(ctx, wo):
    return pl.pallas_call(
        _rs_body,
        out_shape=jax.ShapeDtypeStruct((SQ, D), jnp.float32),
        in_specs=[
            pl.BlockSpec(memory_space=pltpu.VMEM),
            pl.BlockSpec(memory_space=pltpu.VMEM),
        ],
        out_specs=pl.BlockSpec(memory_space=pltpu.VMEM),
        scratch_shapes=[
            pltpu.VMEM((N_DEV - 1, SQ, D), jnp.bfloat16),
            pltpu.VMEM((N_DEV - 1, SQ, D), jnp.bfloat16),
            pltpu.SemaphoreType.DMA((N_DEV - 1, 2)),
            pltpu.SemaphoreType.DMA((N_DEV - 1, 2)),
        ],
        compiler_params=pltpu.CompilerParams(
            collective_id=1, vmem_limit_bytes=62 * 1024 * 1024,
        ),
    )(ctx, wo)


def kernel(x, Wq, Wk, Wv, Wo):
    x2 = x.reshape(SQ, D).astype(jnp.bfloat16)
    wqkv = jnp.concatenate(
        [
            Wq.astype(jnp.bfloat16).reshape(D, H_LOC, DH),
            Wk.astype(jnp.bfloat16).reshape(D, H_LOC, DH),
            Wv.astype(jnp.bfloat16).reshape(D, H_LOC, DH),
        ],
        axis=2,
    ).reshape(D, H_LOC * 3 * DH)
    ctx = _attention(x2, wqkv)
    out = _rs_proj(ctx, Wo.astype(jnp.bfloat16))
    return out.reshape(1, SQ, D)
